# baseline (speedup 1.0000x reference)
import jax
import jax.numpy as jnp
from jax.experimental import pallas as pl
from jax.experimental.pallas import tpu as pltpu

_ROWS = 320


def _dot(a, b):
    return jax.lax.dot_general(
        a, b, (((1,), (0,)), ((), ())),
        precision=jax.lax.Precision.HIGHEST,
        preferred_element_type=jnp.float32)


def _dot_t(a, b):
    return jax.lax.dot_general(
        a, b, (((1,), (1,)), ((), ())),
        precision=jax.lax.Precision.HIGHEST,
        preferred_element_type=jnp.float32)


def _prep_body(x_ref, wt1_ref, bt1_ref, ws1_ref, bs1_ref, xt1_ref, xs1p_ref):
    xb = x_ref[:, 0, :, :]
    xp = jnp.sum(xb, axis=0) * (1.0 / xb.shape[0])
    t1 = _dot(xp, wt1_ref[...]) + bt1_ref[...]
    xt1_ref[...] = jnp.max(t1, axis=0).reshape(1, 1, -1)
    s1 = _dot(xp, ws1_ref[...]) + bs1_ref[...]
    xs1p_ref[...] = s1.reshape(1, *s1.shape)


def _factors_body(xt1_ref, xs1p_ref, wt_ref, ws_ref,
                  es1_ref, es2_ref, et1_ref, et2_ref,
                  xt_ref, adjt_ref, xs2_ref, adjs2_ref):
    n_dim = xs1p_ref.shape[1]
    xt1 = xt1_ref[...]
    xs1 = jnp.max(xs1p_ref[...], axis=0)
    xt_ref[...] = jax.nn.relu(_dot_t(_dot(xt1, wt_ref[...]), xt1))
    xs = jax.nn.relu(_dot_t(_dot(xs1, ws_ref[...]), xs1))
    es1 = es1_ref[...]
    es2 = es2_ref[...]
    adjs = jax.nn.relu(_dot_t(es1, es2) - _dot_t(es2, es1))
    et1 = et1_ref[...]
    et2 = et2_ref[...]
    adjt_ref[...] = jax.nn.relu(_dot_t(et1, et2) - _dot_t(et2, et1))
    xs2_ref[0:n_dim, :] = xs
    xs2_ref[n_dim:2 * n_dim, :] = xs
    adjs2_ref[0:n_dim, :] = adjs
    adjs2_ref[n_dim:2 * n_dim, :] = adjs


def _main_body(xt_ref, adjt_ref, xs2_ref, adjs2_ref, out_ref):
    k = pl.program_id(0)
    s_dim = xt_ref.shape[0]
    n_dim = xs2_ref.shape[1]
    r_blk = out_ref.shape[0]

    r0 = k * r_blk
    row_idx = r0 + jax.lax.broadcasted_iota(jnp.int32, (r_blk, s_dim), 0)
    i_row = row_idx // n_dim
    j_ids = jax.lax.broadcasted_iota(jnp.int32, (r_blk, s_dim), 1)
    onehot = (i_row == j_ids).astype(jnp.float32)
    xti = _dot(onehot, xt_ref[...])
    adjti = _dot(onehot, adjt_ref[...])

    a0 = jax.lax.rem(r0, n_dim)
    base = pl.multiple_of((a0 // 8) * 8, 8)
    rem8 = a0 - base
    w = r_blk + 8
    xsw = xs2_ref[pl.ds(base, w), :]
    adjsw = adjs2_ref[pl.ds(base, w), :]
    xsr = jax.lax.slice(pltpu.roll(xsw, jax.lax.rem(w - rem8, w), 0),
                        (0, 0), (r_blk, n_dim))
    adjsr = jax.lax.slice(pltpu.roll(adjsw, jax.lax.rem(w - rem8, w), 0),
                          (0, 0), (r_blk, n_dim))

    ms = jnp.max(xsr, axis=1, keepdims=True)
    mt = jnp.max(xti, axis=1, keepdims=True)
    m = ms * mt

    denom = jnp.zeros((r_blk, 1), jnp.float32)
    for j in range(s_dim):
        t = jax.lax.slice(xti, (0, j), (r_blk, j + 1))
        aj = jax.lax.slice(adjti, (0, j), (r_blk, j + 1))
        e = jnp.exp(xsr * t - m)
        denom = denom + jnp.sum(e, axis=1, keepdims=True)
        out_ref[:, pl.ds(j * n_dim, n_dim)] = e * (aj * adjsr)
    out_ref[...] = out_ref[...] * (1.0 / denom)


def kernel(x, embs1, embs2, embt1, embt2, Wt1, bt1, Ws1, bs1, wt_, ws_,
           idx_s, idx_t):
    B, S, N, D = x.shape
    M = Wt1.shape[1]

    e_s1 = jnp.take(embs1, idx_s, axis=0)
    e_s2 = jnp.take(embs2, idx_s, axis=0)
    e_t1 = jnp.take(embt1, idx_t, axis=0)
    e_t2 = jnp.take(embt2, idx_t, axis=0)

    xt1, xs1p = pl.pallas_call(
        _prep_body,
        grid=(S,),
        in_specs=[
            pl.BlockSpec((B, 1, N, D), lambda s: (0, s, 0, 0)),
            pl.BlockSpec((D, M), lambda s: (0, 0)),
            pl.BlockSpec((1, M), lambda s: (0, 0)),
            pl.BlockSpec((D, M), lambda s: (0, 0)),
            pl.BlockSpec((1, M), lambda s: (0, 0)),
        ],
        out_specs=[
            pl.BlockSpec((1, 1, M), lambda s: (s, 0, 0)),
            pl.BlockSpec((1, N, M), lambda s: (s, 0, 0)),
        ],
        out_shape=[
            jax.ShapeDtypeStruct((S, 1, M), jnp.float32),
            jax.ShapeDtypeStruct((S, N, M), jnp.float32),
        ],
        compiler_params=pltpu.CompilerParams(
            dimension_semantics=("parallel",)),
    )(x, Wt1, bt1.reshape(1, M), Ws1, bs1.reshape(1, M))

    xt1 = xt1.reshape(S, M)

    xt, adjt, xs2, adjs2 = pl.pallas_call(
        _factors_body,
        out_shape=[
            jax.ShapeDtypeStruct((S, S), jnp.float32),
            jax.ShapeDtypeStruct((S, S), jnp.float32),
            jax.ShapeDtypeStruct((2 * N, N), jnp.float32),
            jax.ShapeDtypeStruct((2 * N, N), jnp.float32),
        ],
    )(xt1, xs1p, wt_, ws_, e_s1, e_s2, e_t1, e_t2)

    n_blocks = (S * N + _ROWS - 1) // _ROWS
    out = pl.pallas_call(
        _main_body,
        grid=(n_blocks,),
        in_specs=[
            pl.BlockSpec((S, S), lambda k: (0, 0)),
            pl.BlockSpec((S, S), lambda k: (0, 0)),
            pl.BlockSpec((2 * N, N), lambda k: (0, 0)),
            pl.BlockSpec((2 * N, N), lambda k: (0, 0)),
        ],
        out_specs=pl.BlockSpec((_ROWS, S * N), lambda k: (k, 0)),
        out_shape=jax.ShapeDtypeStruct((S * N, S * N), jnp.float32),
        compiler_params=pltpu.CompilerParams(
            dimension_semantics=("parallel",)),
    )(xt, adjt, xs2, adjs2)

    return out


# speedup vs baseline: 1.0685x; 1.0685x over previous
import jax
import jax.numpy as jnp
from jax.experimental import pallas as pl
from jax.experimental.pallas import tpu as pltpu

_ROWS = 320


def _dot(a, b):
    return jax.lax.dot_general(
        a, b, (((1,), (0,)), ((), ())),
        precision=jax.lax.Precision.HIGHEST,
        preferred_element_type=jnp.float32)


def _dot_t(a, b):
    return jax.lax.dot_general(
        a, b, (((1,), (1,)), ((), ())),
        precision=jax.lax.Precision.HIGHEST,
        preferred_element_type=jnp.float32)


def _prep_body(x_ref, wt1_ref, bt1_ref, ws1_ref, bs1_ref, xt1_ref, xs1p_ref):
    xb = x_ref[:, 0, :, :]
    xp = jnp.sum(xb, axis=0) * (1.0 / xb.shape[0])
    t1 = _dot(xp, wt1_ref[...]) + bt1_ref[...]
    xt1_ref[...] = jnp.max(t1, axis=0).reshape(1, 1, -1)
    s1 = _dot(xp, ws1_ref[...]) + bs1_ref[...]
    xs1p_ref[...] = s1.reshape(1, *s1.shape)


def _factors_body(xt1_ref, xs1p_ref, wt_ref, ws_ref,
                  es1_ref, es2_ref, et1_ref, et2_ref,
                  xt_ref, adjt_ref, xs2_ref, adjs2_ref):
    n_dim = xs1p_ref.shape[1]
    xt1 = xt1_ref[...]
    xs1 = jnp.max(xs1p_ref[...], axis=0)
    xt_ref[...] = jax.nn.relu(_dot_t(_dot(xt1, wt_ref[...]), xt1))
    xs = jax.nn.relu(_dot_t(_dot(xs1, ws_ref[...]), xs1))
    es1 = es1_ref[...]
    es2 = es2_ref[...]
    adjs = jax.nn.relu(_dot_t(es1, es2) - _dot_t(es2, es1))
    et1 = et1_ref[...]
    et2 = et2_ref[...]
    adjt_ref[...] = jax.nn.relu(_dot_t(et1, et2) - _dot_t(et2, et1))
    xs2_ref[0:n_dim, :] = xs
    xs2_ref[n_dim:2 * n_dim, :] = xs
    adjs2_ref[0:n_dim, :] = adjs
    adjs2_ref[n_dim:2 * n_dim, :] = adjs


def _main_body(xt_ref, adjt_ref, xs2_ref, adjs2_ref, out_ref,
               xsr_s, adjsr_s):
    k = pl.program_id(0)
    s_dim = xt_ref.shape[0]
    n_dim = xs2_ref.shape[1]
    r_blk = out_ref.shape[0]

    r0 = k * r_blk
    row_idx = r0 + jax.lax.broadcasted_iota(jnp.int32, (r_blk, s_dim), 0)
    i_row = row_idx // n_dim
    j_ids = jax.lax.broadcasted_iota(jnp.int32, (r_blk, s_dim), 1)
    onehot = (i_row == j_ids).astype(jnp.float32)
    xti = _dot(onehot, xt_ref[...])
    adjti = _dot(onehot, adjt_ref[...])

    a0 = jax.lax.rem(r0, n_dim)
    base = pl.multiple_of((a0 // 8) * 8, 8)
    rem8 = a0 - base
    w = r_blk + 8
    shift = jax.lax.rem(w - rem8, w)
    xsw = xs2_ref[pl.ds(base, w), :]
    adjsw = adjs2_ref[pl.ds(base, w), :]
    xsr_s[...] = jax.lax.slice(pltpu.roll(xsw, shift, 0),
                               (0, 0), (r_blk, n_dim))
    adjsr_s[...] = jax.lax.slice(pltpu.roll(adjsw, shift, 0),
                                 (0, 0), (r_blk, n_dim))

    ms = jnp.max(xsr_s[...], axis=1, keepdims=True)
    mt = jnp.max(xti, axis=1, keepdims=True)
    m = ms * mt

    denom = jnp.zeros((r_blk, 1), jnp.float32)
    for j in range(s_dim):
        t = jax.lax.slice(xti, (0, j), (r_blk, j + 1))
        denom = denom + jnp.sum(jnp.exp(xsr_s[...] * t - m),
                                axis=1, keepdims=True)
    invd = 1.0 / denom

    for j in range(s_dim):
        t = jax.lax.slice(xti, (0, j), (r_blk, j + 1))
        aj = jax.lax.slice(adjti, (0, j), (r_blk, j + 1))
        e = jnp.exp(xsr_s[...] * t - m)
        out_ref[:, pl.ds(j * n_dim, n_dim)] = e * ((aj * invd) * adjsr_s[...])


def kernel(x, embs1, embs2, embt1, embt2, Wt1, bt1, Ws1, bs1, wt_, ws_,
           idx_s, idx_t):
    B, S, N, D = x.shape
    M = Wt1.shape[1]

    e_s1 = jnp.take(embs1, idx_s, axis=0)
    e_s2 = jnp.take(embs2, idx_s, axis=0)
    e_t1 = jnp.take(embt1, idx_t, axis=0)
    e_t2 = jnp.take(embt2, idx_t, axis=0)

    xt1, xs1p = pl.pallas_call(
        _prep_body,
        grid=(S,),
        in_specs=[
            pl.BlockSpec((B, 1, N, D), lambda s: (0, s, 0, 0)),
            pl.BlockSpec((D, M), lambda s: (0, 0)),
            pl.BlockSpec((1, M), lambda s: (0, 0)),
            pl.BlockSpec((D, M), lambda s: (0, 0)),
            pl.BlockSpec((1, M), lambda s: (0, 0)),
        ],
        out_specs=[
            pl.BlockSpec((1, 1, M), lambda s: (s, 0, 0)),
            pl.BlockSpec((1, N, M), lambda s: (s, 0, 0)),
        ],
        out_shape=[
            jax.ShapeDtypeStruct((S, 1, M), jnp.float32),
            jax.ShapeDtypeStruct((S, N, M), jnp.float32),
        ],
        compiler_params=pltpu.CompilerParams(
            dimension_semantics=("parallel",)),
    )(x, Wt1, bt1.reshape(1, M), Ws1, bs1.reshape(1, M))

    xt1 = xt1.reshape(S, M)

    xt, adjt, xs2, adjs2 = pl.pallas_call(
        _factors_body,
        out_shape=[
            jax.ShapeDtypeStruct((S, S), jnp.float32),
            jax.ShapeDtypeStruct((S, S), jnp.float32),
            jax.ShapeDtypeStruct((2 * N, N), jnp.float32),
            jax.ShapeDtypeStruct((2 * N, N), jnp.float32),
        ],
    )(xt1, xs1p, wt_, ws_, e_s1, e_s2, e_t1, e_t2)

    n_blocks = (S * N + _ROWS - 1) // _ROWS
    out = pl.pallas_call(
        _main_body,
        grid=(n_blocks,),
        in_specs=[
            pl.BlockSpec((S, S), lambda k: (0, 0)),
            pl.BlockSpec((S, S), lambda k: (0, 0)),
            pl.BlockSpec((2 * N, N), lambda k: (0, 0)),
            pl.BlockSpec((2 * N, N), lambda k: (0, 0)),
        ],
        out_specs=pl.BlockSpec((_ROWS, S * N), lambda k: (k, 0)),
        out_shape=jax.ShapeDtypeStruct((S * N, S * N), jnp.float32),
        scratch_shapes=[
            pltpu.VMEM((_ROWS, N), jnp.float32),
            pltpu.VMEM((_ROWS, N), jnp.float32),
        ],
        compiler_params=pltpu.CompilerParams(
            dimension_semantics=("parallel",)),
    )(xt, adjt, xs2, adjs2)

    return out


# speedup vs baseline: 1.1145x; 1.0431x over previous
import jax
import jax.numpy as jnp
from jax.experimental import pallas as pl
from jax.experimental.pallas import tpu as pltpu

_ROWS = 320


def _dot(a, b):
    return jax.lax.dot_general(
        a, b, (((1,), (0,)), ((), ())),
        precision=jax.lax.Precision.HIGHEST,
        preferred_element_type=jnp.float32)


def _dot_t(a, b):
    return jax.lax.dot_general(
        a, b, (((1,), (1,)), ((), ())),
        precision=jax.lax.Precision.HIGHEST,
        preferred_element_type=jnp.float32)


def _prep_body(x_ref, wt1_ref, bt1_ref, ws1_ref, bs1_ref,
               xt1_ref, xs1_ref, acc_s):
    b = pl.program_id(0)
    nb = pl.num_programs(0)
    part = jnp.sum(x_ref[...], axis=0)
    @pl.when(b == 0)
    def _():
        acc_s[...] = part

    @pl.when(b > 0)
    def _():
        acc_s[...] = acc_s[...] + part

    @pl.when(b == nb - 1)
    def _():
        s_dim, n_dim, _ = acc_s.shape
        inv_b = 1.0 / (x_ref.shape[0] * nb)
        t1s = []
        s1s = []
        for s in range(s_dim):
            xp = acc_s[s] * inv_b
            t1 = _dot(xp, wt1_ref[...]) + bt1_ref[...]
            t1s.append(jnp.max(t1, axis=0, keepdims=True))
            s1s.append(_dot(xp, ws1_ref[...]) + bs1_ref[...])
        xt1_ref[...] = jnp.concatenate(t1s, axis=0)
        xs1 = s1s[0]
        for s in range(1, s_dim):
            xs1 = jnp.maximum(xs1, s1s[s])
        xs1_ref[...] = xs1


def _factors_body(xt1_ref, xs1_ref, wt_ref, ws_ref,
                  es1_ref, es2_ref, et1_ref, et2_ref,
                  xt_ref, adjt_ref, xs2_ref, adjs2_ref):
    n_dim = xs1_ref.shape[0]
    xt1 = xt1_ref[...]
    xs1 = xs1_ref[...]
    xt_ref[...] = jax.nn.relu(_dot_t(_dot(xt1, wt_ref[...]), xt1))
    xs = jax.nn.relu(_dot_t(_dot(xs1, ws_ref[...]), xs1))
    es1 = es1_ref[...]
    es2 = es2_ref[...]
    adjs = jax.nn.relu(_dot_t(es1, es2) - _dot_t(es2, es1))
    et1 = et1_ref[...]
    et2 = et2_ref[...]
    adjt_ref[...] = jax.nn.relu(_dot_t(et1, et2) - _dot_t(et2, et1))
    xs2_ref[0:n_dim, :] = xs
    xs2_ref[n_dim:2 * n_dim, :] = xs
    xs2_ref[2 * n_dim:3 * n_dim, :] = xs
    adjs2_ref[0:n_dim, :] = adjs
    adjs2_ref[n_dim:2 * n_dim, :] = adjs
    adjs2_ref[2 * n_dim:3 * n_dim, :] = adjs


def _main_body(xt_ref, adjt_ref, xs2_ref, adjs2_ref, out_ref,
               xsr_s, adjsr_s):
    k = pl.program_id(0)
    s_dim = xt_ref.shape[0]
    n_dim = xs2_ref.shape[1]
    r_blk = out_ref.shape[0]

    r0 = k * r_blk
    row_idx = r0 + jax.lax.broadcasted_iota(jnp.int32, (r_blk, s_dim), 0)
    i_row = row_idx // n_dim
    j_ids = jax.lax.broadcasted_iota(jnp.int32, (r_blk, s_dim), 1)
    onehot = (i_row == j_ids).astype(jnp.float32)
    xti = _dot(onehot, xt_ref[...])
    adjti = _dot(onehot, adjt_ref[...])

    a0 = jax.lax.rem(r0, n_dim)
    base = pl.multiple_of((a0 // 8) * 8, 8)
    rem8 = a0 - base
    w = r_blk + 8
    shift = jax.lax.rem(w - rem8, w)
    xsw = xs2_ref[pl.ds(base, w), :]
    adjsw = adjs2_ref[pl.ds(base, w), :]
    xsr_v = jax.lax.slice(pltpu.roll(xsw, shift, 0),
                          (0, 0), (r_blk, n_dim))
    xsr_s[...] = xsr_v
    adjsr_s[...] = jax.lax.slice(pltpu.roll(adjsw, shift, 0),
                                 (0, 0), (r_blk, n_dim))

    ms = jnp.max(xsr_v, axis=1, keepdims=True)
    mt = jnp.max(xti, axis=1, keepdims=True)
    m = ms * mt
    acc = jnp.zeros((r_blk, n_dim), jnp.float32)
    for j in range(s_dim):
        t = jax.lax.slice(xti, (0, j), (r_blk, j + 1))
        aj = jax.lax.slice(adjti, (0, j), (r_blk, j + 1))
        e = jnp.exp(xsr_s[...] * t - m)
        acc = acc + e
        out_ref[:, pl.ds(j * n_dim, n_dim)] = e * (aj * adjsr_s[...])
    invd = 1.0 / jnp.sum(acc, axis=1, keepdims=True)
    out_ref[...] = out_ref[...] * invd


def kernel(x, embs1, embs2, embt1, embt2, Wt1, bt1, Ws1, bs1, wt_, ws_,
           idx_s, idx_t):
    B, S, N, D = x.shape
    M = Wt1.shape[1]

    e_s1 = jnp.take(embs1, idx_s, axis=0)
    e_s2 = jnp.take(embs2, idx_s, axis=0)
    e_t1 = jnp.take(embt1, idx_t, axis=0)
    e_t2 = jnp.take(embt2, idx_t, axis=0)

    bb = 8
    xt1, xs1 = pl.pallas_call(
        _prep_body,
        grid=(B // bb,),
        in_specs=[
            pl.BlockSpec((bb, S, N, D), lambda b: (b, 0, 0, 0)),
            pl.BlockSpec((D, M), lambda b: (0, 0)),
            pl.BlockSpec((1, M), lambda b: (0, 0)),
            pl.BlockSpec((D, M), lambda b: (0, 0)),
            pl.BlockSpec((1, M), lambda b: (0, 0)),
        ],
        out_specs=[
            pl.BlockSpec((S, M), lambda b: (0, 0)),
            pl.BlockSpec((N, M), lambda b: (0, 0)),
        ],
        out_shape=[
            jax.ShapeDtypeStruct((S, M), jnp.float32),
            jax.ShapeDtypeStruct((N, M), jnp.float32),
        ],
        scratch_shapes=[pltpu.VMEM((S, N, D), jnp.float32)],
        compiler_params=pltpu.CompilerParams(
            dimension_semantics=("arbitrary",)),
    )(x, Wt1, bt1.reshape(1, M), Ws1, bs1.reshape(1, M))

    xt, adjt, xs2, adjs2 = pl.pallas_call(
        _factors_body,
        out_shape=[
            jax.ShapeDtypeStruct((S, S), jnp.float32),
            jax.ShapeDtypeStruct((S, S), jnp.float32),
            jax.ShapeDtypeStruct((3 * N, N), jnp.float32),
            jax.ShapeDtypeStruct((3 * N, N), jnp.float32),
        ],
    )(xt1, xs1, wt_, ws_, e_s1, e_s2, e_t1, e_t2)

    n_blocks = (S * N + _ROWS - 1) // _ROWS
    out = pl.pallas_call(
        _main_body,
        grid=(n_blocks,),
        in_specs=[
            pl.BlockSpec((S, S), lambda k: (0, 0)),
            pl.BlockSpec((S, S), lambda k: (0, 0)),
            pl.BlockSpec((3 * N, N), lambda k: (0, 0)),
            pl.BlockSpec((3 * N, N), lambda k: (0, 0)),
        ],
        out_specs=pl.BlockSpec((_ROWS, S * N), lambda k: (k, 0)),
        out_shape=jax.ShapeDtypeStruct((S * N, S * N), jnp.float32),
        scratch_shapes=[
            pltpu.VMEM((_ROWS, N), jnp.float32),
            pltpu.VMEM((_ROWS, N), jnp.float32),
        ],
        compiler_params=pltpu.CompilerParams(
            dimension_semantics=("parallel",)),
    )(xt, adjt, xs2, adjs2)

    return out


# speedup vs baseline: 1.1352x; 1.0185x over previous
import jax
import jax.numpy as jnp
from jax.experimental import pallas as pl
from jax.experimental.pallas import tpu as pltpu

_ROWS = 320


def _dot(a, b):
    return jax.lax.dot_general(
        a, b, (((1,), (0,)), ((), ())),
        precision=jax.lax.Precision.HIGHEST,
        preferred_element_type=jnp.float32)


def _dot_t(a, b):
    return jax.lax.dot_general(
        a, b, (((1,), (1,)), ((), ())),
        precision=jax.lax.Precision.HIGHEST,
        preferred_element_type=jnp.float32)


def _prep_body(x_ref, wt1_ref, bt1_ref, ws1_ref, bs1_ref, xt1_ref, xs1p_ref):
    xb = x_ref[:, 0, :, :]
    xp = jnp.sum(xb, axis=0) * (1.0 / xb.shape[0])
    t1 = _dot(xp, wt1_ref[...]) + bt1_ref[...]
    xt1_ref[...] = jnp.max(t1, axis=0).reshape(1, 1, -1)
    s1 = _dot(xp, ws1_ref[...]) + bs1_ref[...]
    xs1p_ref[...] = s1.reshape(1, *s1.shape)


def _factors_body(xt1_ref, xs1p_ref, wt_ref, ws_ref,
                  es1_ref, es2_ref, et1_ref, et2_ref,
                  xt_ref, adjt_ref, xs2_ref, adjs2_ref):
    n_dim = xs1p_ref.shape[1]
    xt1 = xt1_ref[...]
    xs1 = jnp.max(xs1p_ref[...], axis=0)
    xt_ref[...] = jax.nn.relu(_dot_t(_dot(xt1, wt_ref[...]), xt1))
    xs = jax.nn.relu(_dot_t(_dot(xs1, ws_ref[...]), xs1))
    es1 = es1_ref[...]
    es2 = es2_ref[...]
    adjs = jax.nn.relu(_dot_t(es1, es2) - _dot_t(es2, es1))
    et1 = et1_ref[...]
    et2 = et2_ref[...]
    adjt_ref[...] = jax.nn.relu(_dot_t(et1, et2) - _dot_t(et2, et1))
    xs2_ref[0:n_dim, :] = xs
    xs2_ref[n_dim:2 * n_dim, :] = xs
    xs2_ref[2 * n_dim:3 * n_dim, :] = xs
    adjs2_ref[0:n_dim, :] = adjs
    adjs2_ref[n_dim:2 * n_dim, :] = adjs
    adjs2_ref[2 * n_dim:3 * n_dim, :] = adjs


def _main_body(xt_ref, adjt_ref, xs2_ref, adjs2_ref, out_ref,
               xsr_s, adjsr_s):
    k = pl.program_id(0)
    s_dim = xt_ref.shape[0]
    n_dim = xs2_ref.shape[1]
    r_blk = out_ref.shape[0]

    r0 = k * r_blk
    row_idx = r0 + jax.lax.broadcasted_iota(jnp.int32, (r_blk, s_dim), 0)
    i_row = row_idx // n_dim
    j_ids = jax.lax.broadcasted_iota(jnp.int32, (r_blk, s_dim), 1)
    onehot = (i_row == j_ids).astype(jnp.float32)
    xti = _dot(onehot, xt_ref[...])
    adjti = _dot(onehot, adjt_ref[...])

    a0 = jax.lax.rem(r0, n_dim)
    base = pl.multiple_of((a0 // 8) * 8, 8)
    rem8 = a0 - base
    w = r_blk + 8
    shift = jax.lax.rem(w - rem8, w)
    xsw = xs2_ref[pl.ds(base, w), :]
    adjsw = adjs2_ref[pl.ds(base, w), :]
    xsr_v = jax.lax.slice(pltpu.roll(xsw, shift, 0),
                          (0, 0), (r_blk, n_dim))
    xsr_s[...] = xsr_v
    adjsr_s[...] = jax.lax.slice(pltpu.roll(adjsw, shift, 0),
                                 (0, 0), (r_blk, n_dim))

    ms = jnp.max(xsr_v, axis=1, keepdims=True)
    mt = jnp.max(xti, axis=1, keepdims=True)
    m = ms * mt
    acc = jnp.zeros((r_blk, n_dim), jnp.float32)
    for j in range(s_dim):
        t = jax.lax.slice(xti, (0, j), (r_blk, j + 1))
        aj = jax.lax.slice(adjti, (0, j), (r_blk, j + 1))
        e = jnp.exp(xsr_s[...] * t - m)
        acc = acc + e
        out_ref[:, pl.ds(j * n_dim, n_dim)] = e * (aj * adjsr_s[...])
    invd = 1.0 / jnp.sum(acc, axis=1, keepdims=True)
    out_ref[...] = out_ref[...] * invd


def kernel(x, embs1, embs2, embt1, embt2, Wt1, bt1, Ws1, bs1, wt_, ws_,
           idx_s, idx_t):
    B, S, N, D = x.shape
    M = Wt1.shape[1]

    e_s1 = jnp.take(embs1, idx_s, axis=0)
    e_s2 = jnp.take(embs2, idx_s, axis=0)
    e_t1 = jnp.take(embt1, idx_t, axis=0)
    e_t2 = jnp.take(embt2, idx_t, axis=0)

    xt1, xs1p = pl.pallas_call(
        _prep_body,
        grid=(S,),
        in_specs=[
            pl.BlockSpec((B, 1, N, D), lambda s: (0, s, 0, 0)),
            pl.BlockSpec((D, M), lambda s: (0, 0)),
            pl.BlockSpec((1, M), lambda s: (0, 0)),
            pl.BlockSpec((D, M), lambda s: (0, 0)),
            pl.BlockSpec((1, M), lambda s: (0, 0)),
        ],
        out_specs=[
            pl.BlockSpec((1, 1, M), lambda s: (s, 0, 0)),
            pl.BlockSpec((1, N, M), lambda s: (s, 0, 0)),
        ],
        out_shape=[
            jax.ShapeDtypeStruct((S, 1, M), jnp.float32),
            jax.ShapeDtypeStruct((S, N, M), jnp.float32),
        ],
        compiler_params=pltpu.CompilerParams(
            dimension_semantics=("parallel",)),
    )(x, Wt1, bt1.reshape(1, M), Ws1, bs1.reshape(1, M))

    xt1 = xt1.reshape(S, M)

    xt, adjt, xs2, adjs2 = pl.pallas_call(
        _factors_body,
        out_shape=[
            jax.ShapeDtypeStruct((S, S), jnp.float32),
            jax.ShapeDtypeStruct((S, S), jnp.float32),
            jax.ShapeDtypeStruct((3 * N, N), jnp.float32),
            jax.ShapeDtypeStruct((3 * N, N), jnp.float32),
        ],
    )(xt1, xs1p, wt_, ws_, e_s1, e_s2, e_t1, e_t2)

    n_blocks = (S * N + _ROWS - 1) // _ROWS
    out = pl.pallas_call(
        _main_body,
        grid=(n_blocks,),
        in_specs=[
            pl.BlockSpec((S, S), lambda k: (0, 0)),
            pl.BlockSpec((S, S), lambda k: (0, 0)),
            pl.BlockSpec((3 * N, N), lambda k: (0, 0)),
            pl.BlockSpec((3 * N, N), lambda k: (0, 0)),
        ],
        out_specs=pl.BlockSpec((_ROWS, S * N), lambda k: (k, 0)),
        out_shape=jax.ShapeDtypeStruct((S * N, S * N), jnp.float32),
        scratch_shapes=[
            pltpu.VMEM((_ROWS, N), jnp.float32),
            pltpu.VMEM((_ROWS, N), jnp.float32),
        ],
        compiler_params=pltpu.CompilerParams(
            dimension_semantics=("parallel",)),
    )(xt, adjt, xs2, adjs2)

    return out
